# baseline (speedup 1.0000x reference)
import functools

import jax
import jax.numpy as jnp
import numpy as np
from jax import lax
from jax.experimental import pallas as pl
from jax.experimental.pallas import tpu as pltpu
from jax.experimental.pallas import tpu_sc as plsc

N = 10000
E = 320000
D = 128
H = 128

NC = 2
NS = 16
NW = NC * NS
EPW = E // NW
CHUNK = 80
NCHUNKS = EPW // CHUNK
N_PAD = 10112
ROWS_PER_TILE = N_PAD // NS

BLK = 1000

_UNPACK_PERM = np.empty(H, dtype=np.int32)
for _j in range(H // 32):
    for _i in range(16):
        _UNPACK_PERM[32 * _j + _i] = 32 * _j + 2 * _i
        _UNPACK_PERM[32 * _j + 16 + _i] = 32 * _j + 2 * _i + 1


def _pre_body(x_ref, w1_ref, w2_ref, ms_ref, md_ref):
    xb = x_ref[...]
    ms_ref[...] = jnp.dot(
        xb, w1_ref[...], preferred_element_type=jnp.float32
    ).astype(jnp.bfloat16)
    md_ref[...] = jnp.dot(
        xb, w2_ref[...], preferred_element_type=jnp.float32
    ).astype(jnp.bfloat16)


def _pre(x, W1, W2):
    return pl.pallas_call(
        _pre_body,
        grid=(N // BLK,),
        in_specs=[
            pl.BlockSpec((BLK, D), lambda i: (i, 0)),
            pl.BlockSpec((D, H), lambda i: (0, 0)),
            pl.BlockSpec((D, H), lambda i: (0, 0)),
        ],
        out_specs=[
            pl.BlockSpec((BLK, H), lambda i: (i, 0)),
            pl.BlockSpec((BLK, H), lambda i: (i, 0)),
        ],
        out_shape=[
            jax.ShapeDtypeStruct((N, H), jnp.bfloat16),
            jax.ShapeDtypeStruct((N, H), jnp.bfloat16),
        ],
    )(x, W1, W2)


def _edge_body(msrc_hbm, mdst_hbm, src_hbm, dst_hbm, out_hbm,
               si0, si1, dg0, dg1, ds0, ds1, ra0, rb0, ra1, rb1, m0, m1, agg_sh,
               sga0, sgb0, sga1, sgb1, ss0, ss1,
               ssi0, ssi1, sdg0, sdg1, sds0, sds1):
    c = lax.axis_index("c")
    s = lax.axis_index("s")
    wid = c * NS + s
    si = (si0, si1)
    dg = (dg0, dg1)
    dsb = (ds0, ds1)
    ra = (ra0, ra1)
    rb = (rb0, rb1)
    mm = (m0, m1)
    sga = (sga0, sga1)
    sgb = (sgb0, sgb1)
    ss = (ss0, ss1)
    ssi = (ssi0, ssi1)
    sdg = (sdg0, sdg1)
    sds = (sds0, sds1)

    zeros16 = jnp.zeros((16,), jnp.float32)

    def zrow(r, carry):
        for j in range(H // 16):
            m0[r, pl.ds(j * 16, 16)] = zeros16
        return carry

    lax.fori_loop(0, CHUNK, zrow, 0)
    r0 = s * ROWS_PER_TILE
    for t in range(ROWS_PER_TILE // CHUNK):
        pltpu.sync_copy(m0, agg_sh.at[pl.ds(r0 + t * CHUNK, CHUNK)])
    rem = ROWS_PER_TILE % CHUNK
    if rem:
        pltpu.sync_copy(
            m0.at[pl.ds(0, rem)],
            agg_sh.at[pl.ds(r0 + (ROWS_PER_TILE // CHUNK) * CHUNK, rem)])
    plsc.subcore_barrier()

    def _idx_copy(hbm, ci, buf, sem):
        off = wid * EPW + ci * CHUNK
        return pltpu.make_async_copy(hbm.at[pl.ds(off, CHUNK)], buf, sem)

    def start_sidx(ci, b):
        _idx_copy(src_hbm, ci, si[b], ssi[b]).start()

    def wait_sidx(ci, b):
        _idx_copy(src_hbm, ci, si[b], ssi[b]).wait()

    def start_dgi(ci, b):
        _idx_copy(dst_hbm, ci, dg[b], sdg[b]).start()

    def wait_dgi(ci, b):
        _idx_copy(dst_hbm, ci, dg[b], sdg[b]).wait()

    def start_dsi(ci, b):
        _idx_copy(dst_hbm, ci, dsb[b], sds[b]).start()

    def wait_dsi(ci, b):
        _idx_copy(dst_hbm, ci, dsb[b], sds[b]).wait()

    def start_gather(b):
        pltpu.make_async_copy(msrc_hbm.at[si[b]], ra[b], sga[b]).start()
        pltpu.make_async_copy(mdst_hbm.at[dg[b]], rb[b], sgb[b]).start()

    def wait_gather(b):
        pltpu.make_async_copy(msrc_hbm.at[si[b]], ra[b], sga[b]).wait()
        pltpu.make_async_copy(mdst_hbm.at[dg[b]], rb[b], sgb[b]).wait()

    def start_scatter(b):
        pltpu.make_async_copy(mm[b], agg_sh.at[dsb[b]], ss[b]).start(add=True)

    def wait_scatter(b):
        pltpu.make_async_copy(mm[b], agg_sh.at[dsb[b]], ss[b]).wait()

    def compute(b):
        A, B, M = ra[b], rb[b], mm[b]
        hi_mask = jnp.full((16,), -65536, jnp.int32)

        def widen(vi):
            ev = lax.bitcast_convert_type(vi << 16, jnp.float32)
            od = lax.bitcast_convert_type(vi & hi_mask, jnp.float32)
            return ev, od

        def rowgrp(r2, carry):
            for rr in range(2):
                r = r2 * 2 + rr
                for j in range(H // 32):
                    ae, ao = widen(A[r, pl.ds(j * 16, 16)])
                    be, bo = widen(B[r, pl.ds(j * 16, 16)])
                    M[r, pl.ds(j * 32, 16)] = jnp.maximum(ae + be, 0.0)
                    M[r, pl.ds(j * 32 + 16, 16)] = jnp.maximum(ao + bo, 0.0)
            return carry

        lax.fori_loop(0, CHUNK // 2, rowgrp, 0)

    start_sidx(0, 0)
    start_sidx(1, 1)
    start_dgi(0, 0)
    start_dgi(1, 1)
    wait_sidx(0, 0)
    wait_sidx(1, 1)
    wait_dgi(0, 0)
    wait_dgi(1, 1)
    start_gather(0)
    start_gather(1)

    def stage(ci, b):
        wait_gather(b)

        @pl.when(ci + 2 < NCHUNKS)
        def _():
            start_sidx(ci + 2, b)
            start_dgi(ci + 2, b)

        @pl.when(ci >= 2)
        def _():
            wait_scatter(b)

        start_dsi(ci, b)
        compute(b)
        wait_dsi(ci, b)
        start_scatter(b)

        @pl.when(ci + 2 < NCHUNKS)
        def _():
            wait_sidx(ci + 2, b)
            wait_dgi(ci + 2, b)
            start_gather(b)

    def g_body(g, carry):
        stage(2 * g, 0)
        stage(2 * g + 1, 1)
        return carry

    lax.fori_loop(0, NCHUNKS // 2, g_body, 0)
    if NCHUNKS % 2:
        stage(NCHUNKS - 1, 0)
        wait_scatter(1)
        wait_scatter(0)
    else:
        wait_scatter(0)
        wait_scatter(1)
    plsc.subcore_barrier()

    pltpu.sync_copy(agg_sh.at[pl.ds(r0, ROWS_PER_TILE)],
                    out_hbm.at[c, pl.ds(r0, ROWS_PER_TILE)])


def _edge_agg(m_src, m_dst, src, dst):
    mesh = plsc.VectorSubcoreMesh(core_axis_name="c", subcore_axis_name="s")
    f = functools.partial(
        pl.kernel,
        mesh=mesh,
        compiler_params=pltpu.CompilerParams(use_tc_tiling_on_sc=False),
        out_type=jax.ShapeDtypeStruct((NC, N_PAD, H), jnp.float32),
        scratch_types=(
            [pltpu.VMEM((CHUNK,), jnp.int32) for _ in range(6)]
            + [pltpu.VMEM((CHUNK, H // 2), jnp.int32) for _ in range(4)]
            + [pltpu.VMEM((CHUNK, H), jnp.float32) for _ in range(2)]
            + [pltpu.VMEM_SHARED((N_PAD, H), jnp.float32)]
            + [pltpu.SemaphoreType.DMA for _ in range(12)]
        ),
    )(_edge_body)
    return f(m_src, m_dst, src, dst)


def _post_body(x_ref, p0_ref, p1_ref, w3_ref, w4_ref, o_ref):
    acc = jnp.dot(x_ref[...], w3_ref[...], preferred_element_type=jnp.float32)
    agg = p0_ref[...] + p1_ref[...]
    acc = acc + jnp.dot(agg, w4_ref[...], preferred_element_type=jnp.float32)
    o_ref[...] = jnp.maximum(acc, 0.0)


def _post(x, p0, p1, W3, W4):
    return pl.pallas_call(
        _post_body,
        grid=(N // BLK,),
        in_specs=[
            pl.BlockSpec((BLK, D), lambda i: (i, 0)),
            pl.BlockSpec((BLK, H), lambda i: (i, 0)),
            pl.BlockSpec((BLK, H), lambda i: (i, 0)),
            pl.BlockSpec((D, H), lambda i: (0, 0)),
            pl.BlockSpec((H, H), lambda i: (0, 0)),
        ],
        out_specs=pl.BlockSpec((BLK, H), lambda i: (i, 0)),
        out_shape=jax.ShapeDtypeStruct((N, H), jnp.float32),
    )(x, p0, p1, W3, W4)


def kernel(x, edge_index, W1, W2, W3, W4):
    ei = edge_index.astype(jnp.int32)
    src = ei[0]
    dst = ei[1]
    m_src, m_dst = _pre(x, W1, W2)
    m_src_i = lax.bitcast_convert_type(
        m_src.reshape(N, H // 2, 2), jnp.int32)
    m_dst_i = lax.bitcast_convert_type(
        m_dst.reshape(N, H // 2, 2), jnp.int32)
    partials = _edge_agg(m_src_i, m_dst_i, src, dst)
    W4p = W4[_UNPACK_PERM, :]
    return _post(x, partials[0, :N], partials[1, :N], W3, W4p)


# speedup vs baseline: 1.6117x; 1.6117x over previous
import functools

import jax
import jax.numpy as jnp
from jax import lax
from jax.experimental import pallas as pl
from jax.experimental.pallas import tpu as pltpu
from jax.experimental.pallas import tpu_sc as plsc

N = 10000
E = 320000
D = 128
H = 128

NC = 2
NS = 16
NW = NC * NS
EPW = E // NW
CHUNK = 40
NCHUNKS = EPW // CHUNK
N_PAD = 10112
ROWS_PER_TILE = N_PAD // NS

BLK = 1000


def _pre_body(x_ref, w1_ref, w2_ref, ms_ref, md_ref):
    xb = x_ref[...]
    ms_ref[...] = jnp.dot(xb, w1_ref[...], preferred_element_type=jnp.float32)
    md_ref[...] = jnp.dot(xb, w2_ref[...], preferred_element_type=jnp.float32)


def _pre(x, W1, W2):
    return pl.pallas_call(
        _pre_body,
        grid=(N // BLK,),
        in_specs=[
            pl.BlockSpec((BLK, D), lambda i: (i, 0)),
            pl.BlockSpec((D, H), lambda i: (0, 0)),
            pl.BlockSpec((D, H), lambda i: (0, 0)),
        ],
        out_specs=[
            pl.BlockSpec((BLK, H), lambda i: (i, 0)),
            pl.BlockSpec((BLK, H), lambda i: (i, 0)),
        ],
        out_shape=[
            jax.ShapeDtypeStruct((N, H), jnp.float32),
            jax.ShapeDtypeStruct((N, H), jnp.float32),
        ],
    )(x, W1, W2)


def _edge_body(msrc_hbm, mdst_hbm, packed_hbm, out_hbm,
               pk, ss0, ss1, sd0, sd1, sc0, sc1,
               ra0, rb0, ra1, rb1, m0, m1, agg_sh,
               sga0, sgb0, sga1, sgb1, ssc0, ssc1, spk):
    c = lax.axis_index("c")
    s = lax.axis_index("s")
    wid = c * NS + s
    ssrc = (ss0, ss1)
    sdst = (sd0, sd1)
    sdsc = (sc0, sc1)
    ra = (ra0, ra1)
    rb = (rb0, rb1)
    mm = (m0, m1)
    sga = (sga0, sga1)
    sgb = (sgb0, sgb1)
    ssc = (ssc0, ssc1)

    pltpu.make_async_copy(
        packed_hbm.at[pl.ds(wid * EPW, EPW)], pk, spk).start()

    zeros16 = jnp.zeros((16,), jnp.float32)

    def zrow(r, carry):
        for j in range(H // 16):
            m0[r, pl.ds(j * 16, 16)] = zeros16
        return carry

    lax.fori_loop(0, CHUNK, zrow, 0)
    r0 = s * ROWS_PER_TILE
    for t in range(ROWS_PER_TILE // CHUNK):
        pltpu.sync_copy(m0, agg_sh.at[pl.ds(r0 + t * CHUNK, CHUNK)])
    rem = ROWS_PER_TILE % CHUNK
    if rem:
        pltpu.sync_copy(
            m0.at[pl.ds(0, rem)],
            agg_sh.at[pl.ds(r0 + (ROWS_PER_TILE // CHUNK) * CHUNK, rem)])
    pltpu.make_async_copy(
        packed_hbm.at[pl.ds(wid * EPW, EPW)], pk, spk).wait()
    plsc.subcore_barrier()

    lo_mask = jnp.full((16,), (1 << 14) - 1, jnp.int32)

    def unpack_idx(ci, b):
        for off in (0, 16, 24):
            v = pk[pl.ds(ci * CHUNK + off, 16)]
            ssrc[b][pl.ds(off, 16)] = v & lo_mask
            sdst[b][pl.ds(off, 16)] = lax.shift_right_logical(v, 14)

    def unpack_dsc(ci, b):
        for off in (0, 16, 24):
            v = pk[pl.ds(ci * CHUNK + off, 16)]
            sdsc[b][pl.ds(off, 16)] = lax.shift_right_logical(v, 14)

    def start_gather(b):
        pltpu.make_async_copy(msrc_hbm.at[ssrc[b]], ra[b], sga[b]).start()
        pltpu.make_async_copy(mdst_hbm.at[sdst[b]], rb[b], sgb[b]).start()

    def wait_gather(b):
        pltpu.make_async_copy(msrc_hbm.at[ssrc[b]], ra[b], sga[b]).wait()
        pltpu.make_async_copy(mdst_hbm.at[sdst[b]], rb[b], sgb[b]).wait()

    def start_scatter(b):
        pltpu.make_async_copy(mm[b], agg_sh.at[sdsc[b]], ssc[b]).start(add=True)

    def wait_scatter(b):
        pltpu.make_async_copy(mm[b], agg_sh.at[sdsc[b]], ssc[b]).wait()

    def compute(b):
        A, B, M = ra[b], rb[b], mm[b]

        def rowgrp(r2, carry):
            for rr in range(2):
                r = r2 * 2 + rr
                for j in range(H // 16):
                    av = A[r, pl.ds(j * 16, 16)]
                    bv = B[r, pl.ds(j * 16, 16)]
                    M[r, pl.ds(j * 16, 16)] = jnp.maximum(av + bv, 0.0)
            return carry

        lax.fori_loop(0, CHUNK // 2, rowgrp, 0)

    unpack_idx(0, 0)
    unpack_idx(1, 1)
    start_gather(0)
    start_gather(1)

    def stage(ci, b):
        wait_gather(b)

        @pl.when(ci >= 2)
        def _():
            wait_scatter(b)

        unpack_dsc(ci, b)
        compute(b)
        start_scatter(b)

        @pl.when(ci + 2 < NCHUNKS)
        def _():
            unpack_idx(ci + 2, b)
            start_gather(b)

    def g_body(g, carry):
        stage(2 * g, 0)
        stage(2 * g + 1, 1)
        return carry

    lax.fori_loop(0, NCHUNKS // 2, g_body, 0)
    if NCHUNKS % 2:
        stage(NCHUNKS - 1, 0)
        wait_scatter(1)
        wait_scatter(0)
    else:
        wait_scatter(0)
        wait_scatter(1)
    plsc.subcore_barrier()

    pltpu.sync_copy(agg_sh.at[pl.ds(r0, ROWS_PER_TILE)],
                    out_hbm.at[c, pl.ds(r0, ROWS_PER_TILE)])


def _edge_agg(m_src, m_dst, packed):
    mesh = plsc.VectorSubcoreMesh(core_axis_name="c", subcore_axis_name="s")
    f = functools.partial(
        pl.kernel,
        mesh=mesh,
        out_type=jax.ShapeDtypeStruct((NC, N_PAD, H), jnp.float32),
        scratch_types=(
            [pltpu.VMEM((EPW,), jnp.int32)]
            + [pltpu.VMEM((CHUNK,), jnp.int32) for _ in range(6)]
            + [pltpu.VMEM((CHUNK, H), jnp.float32) for _ in range(6)]
            + [pltpu.VMEM_SHARED((N_PAD, H), jnp.float32)]
            + [pltpu.SemaphoreType.DMA for _ in range(7)]
        ),
    )(_edge_body)
    return f(m_src, m_dst, packed)


def _post_body(x_ref, p0_ref, p1_ref, w3_ref, w4_ref, o_ref):
    acc = jnp.dot(x_ref[...], w3_ref[...], preferred_element_type=jnp.float32)
    agg = p0_ref[...] + p1_ref[...]
    acc = acc + jnp.dot(agg, w4_ref[...], preferred_element_type=jnp.float32)
    o_ref[...] = jnp.maximum(acc, 0.0)


def _post(x, p0, p1, W3, W4):
    return pl.pallas_call(
        _post_body,
        grid=(N // BLK,),
        in_specs=[
            pl.BlockSpec((BLK, D), lambda i: (i, 0)),
            pl.BlockSpec((BLK, H), lambda i: (i, 0)),
            pl.BlockSpec((BLK, H), lambda i: (i, 0)),
            pl.BlockSpec((D, H), lambda i: (0, 0)),
            pl.BlockSpec((H, H), lambda i: (0, 0)),
        ],
        out_specs=pl.BlockSpec((BLK, H), lambda i: (i, 0)),
        out_shape=jax.ShapeDtypeStruct((N, H), jnp.float32),
    )(x, p0, p1, W3, W4)


def kernel(x, edge_index, W1, W2, W3, W4):
    ei = edge_index.astype(jnp.int32)
    packed = ei[0] | (ei[1] << 14)
    m_src, m_dst = _pre(x, W1, W2)
    partials = _edge_agg(m_src, m_dst, packed)
    return _post(x, partials[0, :N], partials[1, :N], W3, W4)
